# Initial kernel scaffold; baseline (speedup 1.0000x reference)
import jax
import jax.numpy as jnp
from jax.experimental import pallas as pl


def kernel(x, W):
    raise NotImplementedError("write your pallas kernel here")


# speedup vs baseline: 1.3815x; 1.3815x over previous
import functools

import jax
import jax.numpy as jnp
from jax.experimental import pallas as pl

D_MODEL = 768
N_EXPERTS = 64
K = 2
NUM_GATES = 4

BT = 1024


def _router_kernel(x_ref, w_ref, idx_ref, scr_ref, probs_ref, imp_ref, load_ref):
    logits = jax.lax.dot_general(
        x_ref[:], w_ref[:],
        dimension_numbers=(((1,), (1,)), ((), ())),
        preferred_element_type=jnp.float32,
    )
    pacc = None
    for g in range(NUM_GATES):
        lg = logits[:, g * N_EXPERTS:(g + 1) * N_EXPERTS]
        mg = jnp.max(lg, axis=-1, keepdims=True)
        eg = jnp.exp(lg - mg)
        sg = jnp.sum(eg, axis=-1, keepdims=True)
        pg = eg / sg
        pacc = pg if pacc is None else pacc + pg
    probs = pacc * (1.0 / NUM_GATES)
    probs_ref[:] = probs

    iota = jax.lax.broadcasted_iota(jnp.int32, (BT, N_EXPERTS), 1)
    m1 = jnp.max(probs, axis=-1, keepdims=True)
    i1 = jnp.min(jnp.where(probs == m1, iota, N_EXPERTS), axis=-1, keepdims=True)
    masked = jnp.where(iota == i1, -jnp.inf, probs)
    m2 = jnp.max(masked, axis=-1, keepdims=True)
    i2 = jnp.min(jnp.where(masked == m2, iota, N_EXPERTS), axis=-1, keepdims=True)
    den = jnp.maximum(m1 + m2, 1e-9)
    idx_ref[:] = jnp.concatenate([i1, i2], axis=-1)
    scr_ref[:] = jnp.concatenate([m1 / den, m2 / den], axis=-1)

    @pl.when(pl.program_id(0) == 0)
    def _init():
        imp_ref[:] = jnp.zeros_like(imp_ref)
        load_ref[:] = jnp.zeros_like(load_ref)

    psum = jnp.sum(probs, axis=0, keepdims=True)
    lsum = jnp.sum((probs > 0.0).astype(jnp.float32), axis=0, keepdims=True)
    imp_ref[:] += jnp.broadcast_to(psum, imp_ref.shape)
    load_ref[:] += jnp.broadcast_to(lsum, load_ref.shape)


@functools.partial(jax.jit, static_argnames=())
def kernel(x, W):
    B, S, D = x.shape
    T = B * S
    xf = x.reshape(T, D)
    wf = W.reshape(NUM_GATES * N_EXPERTS, D)

    grid = (T // BT,)
    out = pl.pallas_call(
        _router_kernel,
        grid=grid,
        in_specs=[
            pl.BlockSpec((BT, D), lambda i: (i, 0)),
            pl.BlockSpec((NUM_GATES * N_EXPERTS, D), lambda i: (0, 0)),
        ],
        out_specs=[
            pl.BlockSpec((BT, K), lambda i: (i, 0)),
            pl.BlockSpec((BT, K), lambda i: (i, 0)),
            pl.BlockSpec((BT, N_EXPERTS), lambda i: (i, 0)),
            pl.BlockSpec((8, N_EXPERTS), lambda i: (0, 0)),
            pl.BlockSpec((8, N_EXPERTS), lambda i: (0, 0)),
        ],
        out_shape=[
            jax.ShapeDtypeStruct((T, K), jnp.int32),
            jax.ShapeDtypeStruct((T, K), jnp.float32),
            jax.ShapeDtypeStruct((T, N_EXPERTS), jnp.float32),
            jax.ShapeDtypeStruct((8, N_EXPERTS), jnp.float32),
            jax.ShapeDtypeStruct((8, N_EXPERTS), jnp.float32),
        ],
    )(xf, wf)
    idx_f, scr_f, probs_f, imp_acc, load_acc = out
    idx = idx_f.reshape(B, S, K)
    scores = scr_f.reshape(B, S, K)
    probs_full = probs_f.reshape(B, S, N_EXPERTS)
    inv_t = 1.0 / float(T)
    importance = imp_acc[0] * inv_t
    load = load_acc[0] * inv_t
    return (idx, scores, probs_full, importance, load)


# speedup vs baseline: 1.5658x; 1.1335x over previous
import functools

import jax
import jax.numpy as jnp
from jax.experimental import pallas as pl

D_MODEL = 768
N_EXPERTS = 64
K = 2
NUM_GATES = 4

BT = 2048


def _router_kernel(x_ref, w_ref, idx_ref, scr_ref, probs_ref, imp_ref, load_ref):
    logits = jax.lax.dot_general(
        x_ref[:], w_ref[:],
        dimension_numbers=(((1,), (1,)), ((), ())),
        preferred_element_type=jnp.float32,
    )
    pacc = None
    for g in range(NUM_GATES):
        lg = logits[:, g * N_EXPERTS:(g + 1) * N_EXPERTS]
        mg = jnp.max(lg, axis=-1, keepdims=True)
        eg = jnp.exp(lg - mg)
        sg = jnp.sum(eg, axis=-1, keepdims=True)
        pg = eg / sg
        pacc = pg if pacc is None else pacc + pg
    probs = pacc * (1.0 / NUM_GATES)
    probs_ref[:] = probs

    iota = jax.lax.broadcasted_iota(jnp.int32, (BT, N_EXPERTS), 1)
    m1 = jnp.max(probs, axis=-1, keepdims=True)
    i1 = jnp.min(jnp.where(probs == m1, iota, N_EXPERTS), axis=-1, keepdims=True)
    masked = jnp.where(iota == i1, -jnp.inf, probs)
    m2 = jnp.max(masked, axis=-1, keepdims=True)
    i2 = jnp.min(jnp.where(masked == m2, iota, N_EXPERTS), axis=-1, keepdims=True)
    den = jnp.maximum(m1 + m2, 1e-9)
    idx_ref[:] = jnp.concatenate([i1, i2], axis=-1)
    scr_ref[:] = jnp.concatenate([m1 / den, m2 / den], axis=-1)

    @pl.when(pl.program_id(0) == 0)
    def _init():
        imp_ref[:] = jnp.zeros_like(imp_ref)
        load_ref[:] = jnp.zeros_like(load_ref)

    psum = jnp.sum(probs, axis=0, keepdims=True)
    lsum = jnp.sum((probs > 0.0).astype(jnp.float32), axis=0, keepdims=True)
    imp_ref[:] += jnp.broadcast_to(psum, imp_ref.shape)
    load_ref[:] += jnp.broadcast_to(lsum, load_ref.shape)


@functools.partial(jax.jit, static_argnames=())
def kernel(x, W):
    B, S, D = x.shape
    T = B * S
    xf = x.reshape(T, D)
    wf = W.reshape(NUM_GATES * N_EXPERTS, D)

    grid = (T // BT,)
    out = pl.pallas_call(
        _router_kernel,
        grid=grid,
        in_specs=[
            pl.BlockSpec((BT, D), lambda i: (i, 0)),
            pl.BlockSpec((NUM_GATES * N_EXPERTS, D), lambda i: (0, 0)),
        ],
        out_specs=[
            pl.BlockSpec((BT, K), lambda i: (i, 0)),
            pl.BlockSpec((BT, K), lambda i: (i, 0)),
            pl.BlockSpec((BT, N_EXPERTS), lambda i: (i, 0)),
            pl.BlockSpec((8, N_EXPERTS), lambda i: (0, 0)),
            pl.BlockSpec((8, N_EXPERTS), lambda i: (0, 0)),
        ],
        out_shape=[
            jax.ShapeDtypeStruct((T, K), jnp.int32),
            jax.ShapeDtypeStruct((T, K), jnp.float32),
            jax.ShapeDtypeStruct((T, N_EXPERTS), jnp.float32),
            jax.ShapeDtypeStruct((8, N_EXPERTS), jnp.float32),
            jax.ShapeDtypeStruct((8, N_EXPERTS), jnp.float32),
        ],
    )(xf, wf)
    idx_f, scr_f, probs_f, imp_acc, load_acc = out
    idx = idx_f.reshape(B, S, K)
    scores = scr_f.reshape(B, S, K)
    probs_full = probs_f.reshape(B, S, N_EXPERTS)
    inv_t = 1.0 / float(T)
    importance = imp_acc[0] * inv_t
    load = load_acc[0] * inv_t
    return (idx, scores, probs_full, importance, load)


# speedup vs baseline: 1.7957x; 1.1468x over previous
import functools

import jax
import jax.numpy as jnp
from jax.experimental import pallas as pl

D_MODEL = 768
N_EXPERTS = 64
K = 2
NUM_GATES = 4

BT = 4096


def _router_kernel(x_ref, w_ref, idx_ref, scr_ref, probs_ref, imp_ref, load_ref):
    logits = jax.lax.dot_general(
        x_ref[:], w_ref[:],
        dimension_numbers=(((1,), (1,)), ((), ())),
        preferred_element_type=jnp.float32,
    )
    pacc = None
    for g in range(NUM_GATES):
        lg = logits[:, g * N_EXPERTS:(g + 1) * N_EXPERTS]
        mg = jnp.max(lg, axis=-1, keepdims=True)
        eg = jnp.exp(lg - mg)
        sg = jnp.sum(eg, axis=-1, keepdims=True)
        pg = eg / sg
        pacc = pg if pacc is None else pacc + pg
    probs = pacc * (1.0 / NUM_GATES)
    probs_ref[:] = probs

    iota = jax.lax.broadcasted_iota(jnp.int32, (BT, N_EXPERTS), 1)
    m1 = jnp.max(probs, axis=-1, keepdims=True)
    i1 = jnp.min(jnp.where(probs == m1, iota, N_EXPERTS), axis=-1, keepdims=True)
    masked = jnp.where(iota == i1, -jnp.inf, probs)
    m2 = jnp.max(masked, axis=-1, keepdims=True)
    i2 = jnp.min(jnp.where(masked == m2, iota, N_EXPERTS), axis=-1, keepdims=True)
    den = jnp.maximum(m1 + m2, 1e-9)
    idx_ref[:] = jnp.concatenate([i1, i2], axis=-1)
    scr_ref[:] = jnp.concatenate([m1 / den, m2 / den], axis=-1)

    @pl.when(pl.program_id(0) == 0)
    def _init():
        imp_ref[:] = jnp.zeros_like(imp_ref)
        load_ref[:] = jnp.zeros_like(load_ref)

    psum = jnp.sum(probs, axis=0, keepdims=True)
    lsum = jnp.sum((probs > 0.0).astype(jnp.float32), axis=0, keepdims=True)
    imp_ref[:] += jnp.broadcast_to(psum, imp_ref.shape)
    load_ref[:] += jnp.broadcast_to(lsum, load_ref.shape)


@functools.partial(jax.jit, static_argnames=())
def kernel(x, W):
    B, S, D = x.shape
    T = B * S
    xf = x.reshape(T, D)
    wf = W.reshape(NUM_GATES * N_EXPERTS, D)

    grid = (T // BT,)
    out = pl.pallas_call(
        _router_kernel,
        grid=grid,
        in_specs=[
            pl.BlockSpec((BT, D), lambda i: (i, 0)),
            pl.BlockSpec((NUM_GATES * N_EXPERTS, D), lambda i: (0, 0)),
        ],
        out_specs=[
            pl.BlockSpec((BT, K), lambda i: (i, 0)),
            pl.BlockSpec((BT, K), lambda i: (i, 0)),
            pl.BlockSpec((BT, N_EXPERTS), lambda i: (i, 0)),
            pl.BlockSpec((8, N_EXPERTS), lambda i: (0, 0)),
            pl.BlockSpec((8, N_EXPERTS), lambda i: (0, 0)),
        ],
        out_shape=[
            jax.ShapeDtypeStruct((T, K), jnp.int32),
            jax.ShapeDtypeStruct((T, K), jnp.float32),
            jax.ShapeDtypeStruct((T, N_EXPERTS), jnp.float32),
            jax.ShapeDtypeStruct((8, N_EXPERTS), jnp.float32),
            jax.ShapeDtypeStruct((8, N_EXPERTS), jnp.float32),
        ],
    )(xf, wf)
    idx_f, scr_f, probs_f, imp_acc, load_acc = out
    idx = idx_f.reshape(B, S, K)
    scores = scr_f.reshape(B, S, K)
    probs_full = probs_f.reshape(B, S, N_EXPERTS)
    inv_t = 1.0 / float(T)
    importance = imp_acc[0] * inv_t
    load = load_acc[0] * inv_t
    return (idx, scores, probs_full, importance, load)
